# baseline (speedup 1.0000x reference)
import functools

import jax
import jax.numpy as jnp
from jax import lax
from jax.experimental import pallas as pl
from jax.experimental.pallas import tpu as pltpu
from jax.experimental.pallas import tpu_sc as plsc

_EMBED_DIM = 64
_GROUP = 128
_NBUF = 5


def _make_gather(num_groups: int):
  info = plsc.get_sparse_core_info()
  nc, ns = info.num_cores, info.num_subcores
  nw = nc * ns
  assert num_groups % nw == 0
  gpw = num_groups // nw
  assert gpw % _NBUF == 0

  mesh = plsc.VectorSubcoreMesh(core_axis_name="c", subcore_axis_name="s")

  @functools.partial(
      pl.kernel,
      mesh=mesh,
      out_type=jax.ShapeDtypeStruct((num_groups * _GROUP, _EMBED_DIM),
                                    jnp.float32),
      scratch_types=[
          pltpu.VMEM((gpw, _GROUP), jnp.int32),
          pltpu.VMEM((_NBUF, _GROUP, _EMBED_DIM), jnp.float32),
          pltpu.SemaphoreType.DMA,
          pltpu.SemaphoreType.DMA,
      ],
      compiler_params=pltpu.CompilerParams(use_tc_tiling_on_sc=False),
  )
  def gather_kernel(idx_hbm, table_hbm, out_hbm, idx_v, rows_v, gsem, wsem):
    wid = lax.axis_index("s") * nc + lax.axis_index("c")
    g0 = wid * gpw
    pltpu.sync_copy(idx_hbm.at[wid], idx_v)

    def out_base(j):
      g = g0 + j
      return (8 * (g // 64) + g % 8) * 1024 + 128 * ((g % 64) // 8)

    def fire_g(j, b):
      pltpu.async_copy(table_hbm.at[idx_v.at[j]],
                       rows_v.at[b], gsem)

    def wait_g(j, b):
      pltpu.make_async_copy(table_hbm.at[idx_v.at[j]],
                            rows_v.at[b], gsem).wait()

    def fire_wb(j, b):
      pltpu.async_copy(rows_v.at[b],
                       out_hbm.at[pl.ds(out_base(j), _GROUP)], wsem)

    def wait_wb(j, b):
      pltpu.make_async_copy(rows_v.at[b],
                            out_hbm.at[pl.ds(out_base(j), _GROUP)],
                            wsem).wait()

    for b in range(_NBUF):
      fire_g(b, b)

    def step(k, carry):
      for b in range(_NBUF):
        j = _NBUF * k + b
        wait_g(j, b)
        fire_wb(j, b)
        wait_wb(j, b)
        fire_g(j + _NBUF, b)
      return carry

    lax.fori_loop(0, gpw // _NBUF - 1, step, 0)

    for b in range(_NBUF):
      j = gpw - _NBUF + b
      wait_g(j, b)
      fire_wb(j, b)
      wait_wb(j, b)

  return gather_kernel


def kernel(input_ids, word_embeddings):
  batch, seq = input_ids.shape
  vocab, dim = word_embeddings.shape
  n = batch * seq
  num_groups = n // _GROUP
  nw = 32
  idx = (input_ids.T.reshape(seq // 8, 8, batch // _GROUP, _GROUP)
         .transpose(0, 2, 1, 3)
         .reshape(nw, num_groups // nw, _GROUP))
  table = lax.optimization_barrier(
      word_embeddings.reshape(vocab // 2, 2 * dim)).reshape(vocab, dim)
  out = _make_gather(num_groups)(idx, table)
  return out.reshape(seq, batch, dim).transpose(1, 0, 2)
